# Initial kernel scaffold; baseline (speedup 1.0000x reference)
import jax
import jax.numpy as jnp
from jax.experimental import pallas as pl


def kernel(x, edge_index, W1, b1, Wmu, bmu, Wls, bls):
    raise NotImplementedError("write your pallas kernel here")


# speedup vs baseline: 14.6699x; 14.6699x over previous
import functools

import jax
import jax.numpy as jnp
from jax import lax
from jax.experimental import pallas as pl
from jax.experimental.pallas import tpu as pltpu
from jax.experimental.pallas import tpu_sc as plsc

N_NODES = 10000
N_EDGES = 320000
HID = 128
HALF = 64
LAT = 64

NC = 2
NS = 16
CHUNK = 128
NCHUNK = 160
EPAD = NS * NCHUNK * CHUNK
NPAD = 10240
RPT = NPAD // NS
DEG_W = 16
RB = 512


def _sc_mesh():
    return plsc.VectorSubcoreMesh(core_axis_name="c", subcore_axis_name="s")


_SC_PARAMS = pltpu.CompilerParams(use_tc_tiling_on_sc=False)


def _sc_degree(dst3, zeros_deg, ones):
    half_chunks = NCHUNK // NC

    @functools.partial(
        pl.kernel,
        out_type=jax.ShapeDtypeStruct((NC, NPAD, DEG_W), jnp.float32),
        mesh=_sc_mesh(),
        compiler_params=_SC_PARAMS,
        scratch_types=[
            pltpu.VMEM((NCHUNK, CHUNK), jnp.int32),
            pltpu.VMEM((CHUNK, DEG_W), jnp.float32),
            pltpu.VMEM_SHARED((NPAD, DEG_W), jnp.float32),
        ],
    )
    def deg_kernel(dst_hbm, z_hbm, ones_hbm, out_hbm, didx, ones_v, acc):
        c = lax.axis_index("c")
        s = lax.axis_index("s")
        pltpu.sync_copy(z_hbm.at[pl.ds(s * RPT, RPT)], acc.at[pl.ds(s * RPT, RPT)])
        pltpu.sync_copy(dst_hbm.at[s], didx)
        pltpu.sync_copy(ones_hbm, ones_v)
        plsc.subcore_barrier()

        def body(j, carry):
            pltpu.sync_copy(ones_v, acc.at[didx.at[c * half_chunks + j]], add=True)
            return carry

        lax.fori_loop(0, half_chunks, body, 0)
        plsc.subcore_barrier()
        pltpu.sync_copy(acc.at[pl.ds(s * RPT, RPT)],
                        out_hbm.at[c, pl.ds(s * RPT, RPT)])

    return deg_kernel(dst3, zeros_deg, ones)


def _sc_aggregate(feat_split, src3, dst3, zeros_half):
    @functools.partial(
        pl.kernel,
        out_type=jax.ShapeDtypeStruct((NC, NPAD, HALF), jnp.float32),
        mesh=_sc_mesh(),
        compiler_params=_SC_PARAMS,
        scratch_types=[
            pltpu.VMEM((NCHUNK, CHUNK), jnp.int32),
            pltpu.VMEM((NCHUNK, CHUNK), jnp.int32),
            pltpu.VMEM((CHUNK, HALF), jnp.float32),
            pltpu.VMEM((CHUNK, HALF), jnp.float32),
            pltpu.VMEM_SHARED((NPAD, HALF), jnp.float32),
            pltpu.SemaphoreType.DMA,
            pltpu.SemaphoreType.DMA,
        ],
    )
    def agg_kernel(feat_hbm, src_hbm, dst_hbm, z_hbm, out_hbm,
                   sidx, didx, rows0, rows1, acc, sem0, sem1):
        c = lax.axis_index("c")
        s = lax.axis_index("s")
        pltpu.sync_copy(z_hbm.at[pl.ds(s * RPT, RPT)], acc.at[pl.ds(s * RPT, RPT)])
        pltpu.sync_copy(src_hbm.at[s], sidx)
        pltpu.sync_copy(dst_hbm.at[s], didx)
        plsc.subcore_barrier()
        feat_c = feat_hbm.at[c]

        pltpu.async_copy(feat_c.at[sidx.at[0]], rows0, sem0)

        def body(i, carry):
            j = 2 * i
            pltpu.async_copy(feat_c.at[sidx.at[j + 1]], rows1, sem1)
            pltpu.make_async_copy(feat_c.at[sidx.at[j]], rows0, sem0).wait()
            pltpu.sync_copy(rows0, acc.at[didx.at[j]], add=True)
            pltpu.async_copy(feat_c.at[sidx.at[j + 2]], rows0, sem0)
            pltpu.make_async_copy(feat_c.at[sidx.at[j + 1]], rows1, sem1).wait()
            pltpu.sync_copy(rows1, acc.at[didx.at[j + 1]], add=True)
            return carry

        lax.fori_loop(0, NCHUNK // 2 - 1, body, 0)
        j = NCHUNK - 2
        pltpu.async_copy(feat_c.at[sidx.at[j + 1]], rows1, sem1)
        pltpu.make_async_copy(feat_c.at[sidx.at[j]], rows0, sem0).wait()
        pltpu.sync_copy(rows0, acc.at[didx.at[j]], add=True)
        pltpu.make_async_copy(feat_c.at[sidx.at[j + 1]], rows1, sem1).wait()
        pltpu.sync_copy(rows1, acc.at[didx.at[j + 1]], add=True)
        plsc.subcore_barrier()
        pltpu.sync_copy(acc.at[pl.ds(s * RPT, RPT)],
                        out_hbm.at[c, pl.ds(s * RPT, RPT)])

    return agg_kernel(feat_split, src3, dst3, zeros_half)


def _dinv_block(degp_blk):
    deg = degp_blk[0, :, 0:1] + degp_blk[1, :, 0:1] + 1.0
    return lax.rsqrt(deg)


def _tc_xs(xp, W1, degp):
    def body(x_ref, w_ref, dp_ref, o_ref):
        dinv = _dinv_block(dp_ref[...])
        xw = jnp.dot(x_ref[...], w_ref[...],
                     preferred_element_type=jnp.float32) * dinv
        o_ref[0] = xw[:, :HALF]
        o_ref[1] = xw[:, HALF:]

    return pl.pallas_call(
        body,
        grid=(NPAD // RB,),
        in_specs=[
            pl.BlockSpec((RB, HID), lambda i: (i, 0)),
            pl.BlockSpec((HID, HID), lambda i: (0, 0)),
            pl.BlockSpec((NC, RB, DEG_W), lambda i: (0, i, 0)),
        ],
        out_specs=pl.BlockSpec((NC, RB, HALF), lambda i: (0, i, 0)),
        out_shape=jax.ShapeDtypeStruct((NC, NPAD, HALF), jnp.float32),
    )(xp, W1, degp)


def _tc_hs(raw, xs_split, degp, b1):
    def body(r_ref, xs_ref, dp_ref, b_ref, o_ref):
        dinv = _dinv_block(dp_ref[...])
        raw_full = jnp.concatenate([r_ref[0], r_ref[1]], axis=1)
        xs = jnp.concatenate([xs_ref[0], xs_ref[1]], axis=1)
        agg = (raw_full + xs) * dinv + b_ref[...]
        hs = jnp.maximum(agg, 0.0) * dinv
        o_ref[0] = hs[:, :HALF]
        o_ref[1] = hs[:, HALF:]

    return pl.pallas_call(
        body,
        grid=(NPAD // RB,),
        in_specs=[
            pl.BlockSpec((NC, RB, HALF), lambda i: (0, i, 0)),
            pl.BlockSpec((NC, RB, HALF), lambda i: (0, i, 0)),
            pl.BlockSpec((NC, RB, DEG_W), lambda i: (0, i, 0)),
            pl.BlockSpec((1, HID), lambda i: (0, 0)),
        ],
        out_specs=pl.BlockSpec((NC, RB, HALF), lambda i: (0, i, 0)),
        out_shape=jax.ShapeDtypeStruct((NC, NPAD, HALF), jnp.float32),
    )(raw, xs_split, degp, b1)


def _tc_out(raw, hs_split, degp, Wcat, bcat):
    def body(r_ref, hs_ref, dp_ref, w_ref, b_ref, o_ref):
        dinv = _dinv_block(dp_ref[...])
        raw_full = jnp.concatenate([r_ref[0], r_ref[1]], axis=1)
        hs = jnp.concatenate([hs_ref[0], hs_ref[1]], axis=1)
        z = (raw_full + hs) * dinv
        o_ref[...] = jnp.dot(z, w_ref[...],
                             preferred_element_type=jnp.float32) + b_ref[...]

    return pl.pallas_call(
        body,
        grid=(NPAD // RB,),
        in_specs=[
            pl.BlockSpec((NC, RB, HALF), lambda i: (0, i, 0)),
            pl.BlockSpec((NC, RB, HALF), lambda i: (0, i, 0)),
            pl.BlockSpec((NC, RB, DEG_W), lambda i: (0, i, 0)),
            pl.BlockSpec((HID, 2 * LAT), lambda i: (0, 0)),
            pl.BlockSpec((1, 2 * LAT), lambda i: (0, 0)),
        ],
        out_specs=pl.BlockSpec((RB, 2 * LAT), lambda i: (i, 0)),
        out_shape=jax.ShapeDtypeStruct((NPAD, 2 * LAT), jnp.float32),
    )(raw, hs_split, degp, Wcat, bcat)


def kernel(x, edge_index, W1, b1, Wmu, bmu, Wls, bls):
    f32 = jnp.float32
    e32 = edge_index.astype(jnp.int32)
    pad = jnp.full((2, EPAD - N_EDGES), N_NODES, jnp.int32)
    e = jnp.concatenate([e32, pad], axis=1)
    src3 = e[0].reshape(NS, NCHUNK, CHUNK)
    dst3 = e[1].reshape(NS, NCHUNK, CHUNK)

    xp = jnp.concatenate([x.astype(f32),
                          jnp.zeros((NPAD - N_NODES, HID), f32)], axis=0)
    zeros_deg = jnp.zeros((NPAD, DEG_W), f32)
    zeros_half = jnp.zeros((NPAD, HALF), f32)
    ones = jnp.ones((CHUNK, DEG_W), f32)

    degp = _sc_degree(dst3, zeros_deg, ones)
    xs_split = _tc_xs(xp, W1, degp)
    raw1 = _sc_aggregate(xs_split, src3, dst3, zeros_half)
    hs_split = _tc_hs(raw1, xs_split, degp, b1.reshape(1, HID))
    raw2 = _sc_aggregate(hs_split, src3, dst3, zeros_half)
    Wcat = jnp.concatenate([Wmu, Wls], axis=1)
    bcat = jnp.concatenate([bmu, bls]).reshape(1, 2 * LAT)
    zc = _tc_out(raw2, hs_split, degp, Wcat, bcat)
    return zc[:N_NODES, :LAT], zc[:N_NODES, LAT:]


# speedup vs baseline: 14.6748x; 1.0003x over previous
import functools

import jax
import jax.numpy as jnp
from jax import lax
from jax.experimental import pallas as pl
from jax.experimental.pallas import tpu as pltpu
from jax.experimental.pallas import tpu_sc as plsc

N_NODES = 10000
N_EDGES = 320000
HID = 128
HALF = 64
LAT = 64

NC = 2
NS = 16
CHUNK = 128
NCHUNK = 160
EPAD = NS * NCHUNK * CHUNK
NPAD = 10240
RPT = NPAD // NS
DEG_W = 16
NBUF = 4
RB = 512


def _sc_mesh():
    return plsc.VectorSubcoreMesh(core_axis_name="c", subcore_axis_name="s")


_SC_PARAMS = pltpu.CompilerParams(use_tc_tiling_on_sc=False)


def _sc_degree(dst3, zeros_deg, ones):
    half_chunks = NCHUNK // NC

    @functools.partial(
        pl.kernel,
        out_type=jax.ShapeDtypeStruct((NC, NPAD, DEG_W), jnp.float32),
        mesh=_sc_mesh(),
        compiler_params=_SC_PARAMS,
        scratch_types=[
            pltpu.VMEM((NCHUNK, CHUNK), jnp.int32),
            pltpu.VMEM((CHUNK, DEG_W), jnp.float32),
            pltpu.VMEM_SHARED((NPAD, DEG_W), jnp.float32),
        ],
    )
    def deg_kernel(dst_hbm, z_hbm, ones_hbm, out_hbm, didx, ones_v, acc):
        c = lax.axis_index("c")
        s = lax.axis_index("s")
        pltpu.sync_copy(z_hbm.at[pl.ds(s * RPT, RPT)], acc.at[pl.ds(s * RPT, RPT)])
        pltpu.sync_copy(dst_hbm.at[s], didx)
        pltpu.sync_copy(ones_hbm, ones_v)
        plsc.subcore_barrier()

        def body(j, carry):
            pltpu.sync_copy(ones_v, acc.at[didx.at[c * half_chunks + j]], add=True)
            return carry

        lax.fori_loop(0, half_chunks, body, 0)
        plsc.subcore_barrier()
        pltpu.sync_copy(acc.at[pl.ds(s * RPT, RPT)],
                        out_hbm.at[c, pl.ds(s * RPT, RPT)])

    return deg_kernel(dst3, zeros_deg, ones)


def _sc_aggregate(feat_split, src3, dst3, zeros_half):
    @functools.partial(
        pl.kernel,
        out_type=jax.ShapeDtypeStruct((NC, NPAD, HALF), jnp.float32),
        mesh=_sc_mesh(),
        compiler_params=_SC_PARAMS,
        scratch_types=[
            pltpu.VMEM((NCHUNK, CHUNK), jnp.int32),
            pltpu.VMEM((NCHUNK, CHUNK), jnp.int32),
            [pltpu.VMEM((CHUNK, HALF), jnp.float32) for _ in range(NBUF)],
            pltpu.VMEM_SHARED((NPAD, HALF), jnp.float32),
            [pltpu.SemaphoreType.DMA for _ in range(NBUF)],
            [pltpu.SemaphoreType.DMA for _ in range(NBUF)],
        ],
    )
    def agg_kernel(feat_hbm, src_hbm, dst_hbm, z_hbm, out_hbm,
                   sidx, didx, rows, acc, gsem, ssem):
        c = lax.axis_index("c")
        s = lax.axis_index("s")
        pltpu.sync_copy(z_hbm.at[pl.ds(s * RPT, RPT)], acc.at[pl.ds(s * RPT, RPT)])
        pltpu.sync_copy(src_hbm.at[s], sidx)
        pltpu.sync_copy(dst_hbm.at[s], didx)
        plsc.subcore_barrier()
        feat_c = feat_hbm.at[c]

        for b in range(NBUF):
            pltpu.async_copy(feat_c.at[sidx.at[b]], rows[b], gsem[b])
        for b in range(NBUF):
            pltpu.make_async_copy(feat_c.at[sidx.at[b]], rows[b], gsem[b]).wait()
            pltpu.async_copy(rows[b], acc.at[didx.at[b]], ssem[b], add=True)

        def body(g, carry):
            j0 = NBUF * g
            for b in range(NBUF):
                pltpu.make_async_copy(rows[b], acc.at[didx.at[j0 + b - NBUF]],
                                      ssem[b]).wait()
                pltpu.async_copy(feat_c.at[sidx.at[j0 + b]], rows[b], gsem[b])
            for b in range(NBUF):
                pltpu.make_async_copy(feat_c.at[sidx.at[j0 + b]], rows[b],
                                      gsem[b]).wait()
                pltpu.async_copy(rows[b], acc.at[didx.at[j0 + b]], ssem[b],
                                 add=True)
            return carry

        lax.fori_loop(1, NCHUNK // NBUF, body, 0)
        for b in range(NBUF):
            pltpu.make_async_copy(rows[b], acc.at[didx.at[NCHUNK - NBUF + b]],
                                  ssem[b]).wait()
        plsc.subcore_barrier()
        pltpu.sync_copy(acc.at[pl.ds(s * RPT, RPT)],
                        out_hbm.at[c, pl.ds(s * RPT, RPT)])

    return agg_kernel(feat_split, src3, dst3, zeros_half)


def _dinv_block(degp_blk):
    deg = degp_blk[0, :, 0:1] + degp_blk[1, :, 0:1] + 1.0
    return lax.rsqrt(deg)


def _tc_xs(xp, W1, degp):
    def body(x_ref, w_ref, dp_ref, o_ref):
        dinv = _dinv_block(dp_ref[...])
        xw = jnp.dot(x_ref[...], w_ref[...],
                     preferred_element_type=jnp.float32) * dinv
        o_ref[0] = xw[:, :HALF]
        o_ref[1] = xw[:, HALF:]

    return pl.pallas_call(
        body,
        grid=(NPAD // RB,),
        in_specs=[
            pl.BlockSpec((RB, HID), lambda i: (i, 0)),
            pl.BlockSpec((HID, HID), lambda i: (0, 0)),
            pl.BlockSpec((NC, RB, DEG_W), lambda i: (0, i, 0)),
        ],
        out_specs=pl.BlockSpec((NC, RB, HALF), lambda i: (0, i, 0)),
        out_shape=jax.ShapeDtypeStruct((NC, NPAD, HALF), jnp.float32),
    )(xp, W1, degp)


def _tc_hs(raw, xs_split, degp, b1):
    def body(r_ref, xs_ref, dp_ref, b_ref, o_ref):
        dinv = _dinv_block(dp_ref[...])
        raw_full = jnp.concatenate([r_ref[0], r_ref[1]], axis=1)
        xs = jnp.concatenate([xs_ref[0], xs_ref[1]], axis=1)
        agg = (raw_full + xs) * dinv + b_ref[...]
        hs = jnp.maximum(agg, 0.0) * dinv
        o_ref[0] = hs[:, :HALF]
        o_ref[1] = hs[:, HALF:]

    return pl.pallas_call(
        body,
        grid=(NPAD // RB,),
        in_specs=[
            pl.BlockSpec((NC, RB, HALF), lambda i: (0, i, 0)),
            pl.BlockSpec((NC, RB, HALF), lambda i: (0, i, 0)),
            pl.BlockSpec((NC, RB, DEG_W), lambda i: (0, i, 0)),
            pl.BlockSpec((1, HID), lambda i: (0, 0)),
        ],
        out_specs=pl.BlockSpec((NC, RB, HALF), lambda i: (0, i, 0)),
        out_shape=jax.ShapeDtypeStruct((NC, NPAD, HALF), jnp.float32),
    )(raw, xs_split, degp, b1)


def _tc_out(raw, hs_split, degp, Wcat, bcat):
    def body(r_ref, hs_ref, dp_ref, w_ref, b_ref, o_ref):
        dinv = _dinv_block(dp_ref[...])
        raw_full = jnp.concatenate([r_ref[0], r_ref[1]], axis=1)
        hs = jnp.concatenate([hs_ref[0], hs_ref[1]], axis=1)
        z = (raw_full + hs) * dinv
        o_ref[...] = jnp.dot(z, w_ref[...],
                             preferred_element_type=jnp.float32) + b_ref[...]

    return pl.pallas_call(
        body,
        grid=(NPAD // RB,),
        in_specs=[
            pl.BlockSpec((NC, RB, HALF), lambda i: (0, i, 0)),
            pl.BlockSpec((NC, RB, HALF), lambda i: (0, i, 0)),
            pl.BlockSpec((NC, RB, DEG_W), lambda i: (0, i, 0)),
            pl.BlockSpec((HID, 2 * LAT), lambda i: (0, 0)),
            pl.BlockSpec((1, 2 * LAT), lambda i: (0, 0)),
        ],
        out_specs=pl.BlockSpec((RB, 2 * LAT), lambda i: (i, 0)),
        out_shape=jax.ShapeDtypeStruct((NPAD, 2 * LAT), jnp.float32),
    )(raw, hs_split, degp, Wcat, bcat)


def kernel(x, edge_index, W1, b1, Wmu, bmu, Wls, bls):
    f32 = jnp.float32
    e32 = edge_index.astype(jnp.int32)
    pad = jnp.full((2, EPAD - N_EDGES), N_NODES, jnp.int32)
    e = jnp.concatenate([e32, pad], axis=1)
    src3 = e[0].reshape(NS, NCHUNK, CHUNK)
    dst3 = e[1].reshape(NS, NCHUNK, CHUNK)

    xp = jnp.concatenate([x.astype(f32),
                          jnp.zeros((NPAD - N_NODES, HID), f32)], axis=0)
    zeros_deg = jnp.zeros((NPAD, DEG_W), f32)
    zeros_half = jnp.zeros((NPAD, HALF), f32)
    ones = jnp.ones((CHUNK, DEG_W), f32)

    degp = _sc_degree(dst3, zeros_deg, ones)
    xs_split = _tc_xs(xp, W1, degp)
    raw1 = _sc_aggregate(xs_split, src3, dst3, zeros_half)
    hs_split = _tc_hs(raw1, xs_split, degp, b1.reshape(1, HID))
    raw2 = _sc_aggregate(hs_split, src3, dst3, zeros_half)
    Wcat = jnp.concatenate([Wmu, Wls], axis=1)
    bcat = jnp.concatenate([bmu, bls]).reshape(1, 2 * LAT)
    zc = _tc_out(raw2, hs_split, degp, Wcat, bcat)
    return zc[:N_NODES, :LAT], zc[:N_NODES, LAT:]


# speedup vs baseline: 25.6231x; 1.7461x over previous
import functools

import jax
import jax.numpy as jnp
from jax import lax
from jax.experimental import pallas as pl
from jax.experimental.pallas import tpu as pltpu
from jax.experimental.pallas import tpu_sc as plsc

N_NODES = 10000
N_EDGES = 320000
HID = 128
HALF = 64
LAT = 64

NC = 2
NS = 16
CHUNK = 128
NCHUNK = 160
EPAD = NS * NCHUNK * CHUNK
NPAD = 10240
RPT = NPAD // NS
DEG_W = 16
NBUF = 2
NCHUNK_SRC = NCHUNK + 2 * NBUF
RB = 512


def _sc_mesh():
    return plsc.VectorSubcoreMesh(core_axis_name="c", subcore_axis_name="s")


_SC_PARAMS = pltpu.CompilerParams(use_tc_tiling_on_sc=False)


def _sc_degree(dst3, zeros_deg, ones):
    half_chunks = NCHUNK // NC

    @functools.partial(
        pl.kernel,
        out_type=jax.ShapeDtypeStruct((NC, NPAD, DEG_W), jnp.float32),
        mesh=_sc_mesh(),
        compiler_params=_SC_PARAMS,
        scratch_types=[
            pltpu.VMEM((NCHUNK, CHUNK), jnp.int32),
            pltpu.VMEM((CHUNK, DEG_W), jnp.float32),
            pltpu.VMEM_SHARED((NPAD, DEG_W), jnp.float32),
        ],
    )
    def deg_kernel(dst_hbm, z_hbm, ones_hbm, out_hbm, didx, ones_v, acc):
        c = lax.axis_index("c")
        s = lax.axis_index("s")
        pltpu.sync_copy(z_hbm.at[pl.ds(s * RPT, RPT)], acc.at[pl.ds(s * RPT, RPT)])
        pltpu.sync_copy(dst_hbm.at[s], didx)
        pltpu.sync_copy(ones_hbm, ones_v)
        plsc.subcore_barrier()

        def body(j, carry):
            pltpu.sync_copy(ones_v, acc.at[didx.at[c * half_chunks + j]], add=True)
            return carry

        lax.fori_loop(0, half_chunks, body, 0)
        plsc.subcore_barrier()
        pltpu.sync_copy(acc.at[pl.ds(s * RPT, RPT)],
                        out_hbm.at[c, pl.ds(s * RPT, RPT)])

    return deg_kernel(dst3, zeros_deg, ones)


def _sc_aggregate(feat_split, src3, dst3, zeros_half):
    @functools.partial(
        pl.kernel,
        out_type=jax.ShapeDtypeStruct((NC, NPAD, HALF), jnp.float32),
        mesh=_sc_mesh(),
        compiler_params=_SC_PARAMS,
        scratch_types=[
            [pltpu.VMEM((NBUF, CHUNK), jnp.int32) for _ in range(2)],
            pltpu.VMEM((NCHUNK, CHUNK), jnp.int32),
            [pltpu.VMEM((CHUNK, HALF), jnp.float32) for _ in range(NBUF)],
            pltpu.VMEM_SHARED((NPAD, HALF), jnp.float32),
            pltpu.VMEM_SHARED((NPAD, HALF), jnp.float32),
            [pltpu.SemaphoreType.DMA for _ in range(2)],
            [pltpu.SemaphoreType.DMA for _ in range(NBUF)],
            [pltpu.SemaphoreType.DMA for _ in range(NBUF)],
        ],
    )
    def agg_kernel(feat_hbm, src_hbm, dst_hbm, z_hbm, out_hbm,
                   sring, didx, rows, feat_s, acc, isem, gsem, ssem):
        c = lax.axis_index("c")
        s = lax.axis_index("s")
        pltpu.sync_copy(z_hbm.at[pl.ds(s * RPT, RPT)], acc.at[pl.ds(s * RPT, RPT)])
        pltpu.sync_copy(feat_hbm.at[c, pl.ds(s * RPT, RPT)],
                        feat_s.at[pl.ds(s * RPT, RPT)])
        pltpu.sync_copy(dst_hbm.at[s], didx)
        plsc.subcore_barrier()

        def prefetch(slot, g):
            pltpu.async_copy(src_hbm.at[s, pl.ds(g * NBUF, NBUF)],
                             sring[slot], isem[slot])

        def wait_prefetch(slot):
            pltpu.make_async_copy(src_hbm.at[s, pl.ds(0, NBUF)],
                                  sring[slot], isem[slot]).wait()

        def do_group(slot, g, first):
            wait_prefetch(slot)
            for b in range(NBUF):
                if not first:
                    pltpu.make_async_copy(rows[b], acc.at[didx.at[0]],
                                          ssem[b]).wait()
                pltpu.async_copy(feat_s.at[sring[slot].at[b]], rows[b], gsem[b])
            for b in range(NBUF):
                pltpu.make_async_copy(feat_s.at[sring[slot].at[b]], rows[b],
                                      gsem[b]).wait()
                pltpu.async_copy(rows[b], acc.at[didx.at[g * NBUF + b]],
                                 ssem[b], add=True)
            prefetch(slot, g + 2)

        prefetch(0, 0)
        prefetch(1, 1)
        do_group(0, 0, True)
        do_group(1, 1, False)

        def body(i, carry):
            do_group(0, 2 * i, False)
            do_group(1, 2 * i + 1, False)
            return carry

        lax.fori_loop(1, NCHUNK // NBUF // 2, body, 0)
        for b in range(NBUF):
            pltpu.make_async_copy(rows[b], acc.at[didx.at[0]], ssem[b]).wait()
        for slot in range(2):
            wait_prefetch(slot)
        plsc.subcore_barrier()
        pltpu.sync_copy(acc.at[pl.ds(s * RPT, RPT)],
                        out_hbm.at[c, pl.ds(s * RPT, RPT)])

    return agg_kernel(feat_split, src3, dst3, zeros_half)


def _dinv_block(degp_blk):
    deg = degp_blk[0, :, 0:1] + degp_blk[1, :, 0:1] + 1.0
    return lax.rsqrt(deg)


def _tc_xs(xp, W1, degp):
    def body(x_ref, w_ref, dp_ref, o_ref):
        dinv = _dinv_block(dp_ref[...])
        xw = jnp.dot(x_ref[...], w_ref[...],
                     preferred_element_type=jnp.float32) * dinv
        o_ref[0] = xw[:, :HALF]
        o_ref[1] = xw[:, HALF:]

    return pl.pallas_call(
        body,
        grid=(NPAD // RB,),
        in_specs=[
            pl.BlockSpec((RB, HID), lambda i: (i, 0)),
            pl.BlockSpec((HID, HID), lambda i: (0, 0)),
            pl.BlockSpec((NC, RB, DEG_W), lambda i: (0, i, 0)),
        ],
        out_specs=pl.BlockSpec((NC, RB, HALF), lambda i: (0, i, 0)),
        out_shape=jax.ShapeDtypeStruct((NC, NPAD, HALF), jnp.float32),
    )(xp, W1, degp)


def _tc_hs(raw, xs_split, degp, b1):
    def body(r_ref, xs_ref, dp_ref, b_ref, o_ref):
        dinv = _dinv_block(dp_ref[...])
        raw_full = jnp.concatenate([r_ref[0], r_ref[1]], axis=1)
        xs = jnp.concatenate([xs_ref[0], xs_ref[1]], axis=1)
        agg = (raw_full + xs) * dinv + b_ref[...]
        hs = jnp.maximum(agg, 0.0) * dinv
        o_ref[0] = hs[:, :HALF]
        o_ref[1] = hs[:, HALF:]

    return pl.pallas_call(
        body,
        grid=(NPAD // RB,),
        in_specs=[
            pl.BlockSpec((NC, RB, HALF), lambda i: (0, i, 0)),
            pl.BlockSpec((NC, RB, HALF), lambda i: (0, i, 0)),
            pl.BlockSpec((NC, RB, DEG_W), lambda i: (0, i, 0)),
            pl.BlockSpec((1, HID), lambda i: (0, 0)),
        ],
        out_specs=pl.BlockSpec((NC, RB, HALF), lambda i: (0, i, 0)),
        out_shape=jax.ShapeDtypeStruct((NC, NPAD, HALF), jnp.float32),
    )(raw, xs_split, degp, b1)


def _tc_out(raw, hs_split, degp, Wcat, bcat):
    def body(r_ref, hs_ref, dp_ref, w_ref, b_ref, o_ref):
        dinv = _dinv_block(dp_ref[...])
        raw_full = jnp.concatenate([r_ref[0], r_ref[1]], axis=1)
        hs = jnp.concatenate([hs_ref[0], hs_ref[1]], axis=1)
        z = (raw_full + hs) * dinv
        o_ref[...] = jnp.dot(z, w_ref[...],
                             preferred_element_type=jnp.float32) + b_ref[...]

    return pl.pallas_call(
        body,
        grid=(NPAD // RB,),
        in_specs=[
            pl.BlockSpec((NC, RB, HALF), lambda i: (0, i, 0)),
            pl.BlockSpec((NC, RB, HALF), lambda i: (0, i, 0)),
            pl.BlockSpec((NC, RB, DEG_W), lambda i: (0, i, 0)),
            pl.BlockSpec((HID, 2 * LAT), lambda i: (0, 0)),
            pl.BlockSpec((1, 2 * LAT), lambda i: (0, 0)),
        ],
        out_specs=pl.BlockSpec((RB, 2 * LAT), lambda i: (i, 0)),
        out_shape=jax.ShapeDtypeStruct((NPAD, 2 * LAT), jnp.float32),
    )(raw, hs_split, degp, Wcat, bcat)


def kernel(x, edge_index, W1, b1, Wmu, bmu, Wls, bls):
    f32 = jnp.float32
    e32 = jnp.concatenate(
        [edge_index.astype(jnp.int32),
         jnp.full((2, EPAD - N_EDGES), N_NODES, jnp.int32)], axis=1)
    dst3 = e32[1].reshape(NS, NCHUNK, CHUNK)
    src3 = jnp.concatenate(
        [e32[0].reshape(NS, NCHUNK, CHUNK),
         jnp.full((NS, NCHUNK_SRC - NCHUNK, CHUNK), N_NODES, jnp.int32)],
        axis=1)

    xp = jnp.concatenate([x.astype(f32),
                          jnp.zeros((NPAD - N_NODES, HID), f32)], axis=0)
    zeros_deg = jnp.zeros((NPAD, DEG_W), f32)
    zeros_half = jnp.zeros((NPAD, HALF), f32)
    ones = jnp.ones((CHUNK, DEG_W), f32)

    degp = _sc_degree(dst3, zeros_deg, ones)
    xs_split = _tc_xs(xp, W1, degp)
    raw1 = _sc_aggregate(xs_split, src3, dst3, zeros_half)
    hs_split = _tc_hs(raw1, xs_split, degp, b1.reshape(1, HID))
    raw2 = _sc_aggregate(hs_split, src3, dst3, zeros_half)
    Wcat = jnp.concatenate([Wmu, Wls], axis=1)
    bcat = jnp.concatenate([bmu, bls]).reshape(1, 2 * LAT)
    zc = _tc_out(raw2, hs_split, degp, Wcat, bcat)
    return zc[:N_NODES, :LAT], zc[:N_NODES, LAT:]
